# baseline (speedup 1.0000x reference)
import functools

import jax
import jax.numpy as jnp
from jax.experimental import pallas as pl

_SMOOTHING = 0.1
_NUM_CLASSES = 1000
_CONFIDENCE = 1.0 - _SMOOTHING
_SMOOTH_VAL = _SMOOTHING / (_NUM_CLASSES - 1)


def _loss_body(x_ref, t_ref, out_ref, *, rows, k):
    i = pl.program_id(0)
    x = x_ref[...]
    m = jnp.max(x, axis=1, keepdims=True)
    e = jnp.exp(x - m)
    lse = jnp.log(jnp.sum(e, axis=1, keepdims=True))
    sum_x = jnp.sum(x, axis=1, keepdims=True)
    sum_logp = sum_x - float(k) * (m + lse)

    t = t_ref[0, 0, :]
    col = jax.lax.broadcasted_iota(jnp.int32, (rows, k), 1)
    p_t = jnp.sum(jnp.where(col == t[:, None], x, 0.0), axis=1, keepdims=True)
    logp_t = p_t - m - lse

    row_loss = -(_SMOOTH_VAL * sum_logp + (_CONFIDENCE - _SMOOTH_VAL) * logp_t)
    partial = jnp.sum(row_loss).reshape(1, 1)

    @pl.when(i == 0)
    def _init():
        out_ref[...] = partial

    @pl.when(i != 0)
    def _acc():
        out_ref[...] += partial


def kernel(pred, target):
    n, k = pred.shape
    rows = 512
    num_blocks = n // rows
    t3 = target.astype(jnp.int32).reshape(num_blocks, 1, rows)

    total = pl.pallas_call(
        functools.partial(_loss_body, rows=rows, k=k),
        grid=(num_blocks,),
        in_specs=[
            pl.BlockSpec((rows, k), lambda i: (i, 0)),
            pl.BlockSpec((1, 1, rows), lambda i: (i, 0, 0)),
        ],
        out_specs=pl.BlockSpec((1, 1), lambda i: (0, 0)),
        out_shape=jax.ShapeDtypeStruct((1, 1), jnp.float32),
    )(pred, t3)
    return (total[0, 0] / n).astype(jnp.float32)


# speedup vs baseline: 1.1251x; 1.1251x over previous
import functools

import jax
import jax.numpy as jnp
from jax.experimental import pallas as pl

_SMOOTHING = 0.1
_NUM_CLASSES = 1000
_CONFIDENCE = 1.0 - _SMOOTHING
_SMOOTH_VAL = _SMOOTHING / (_NUM_CLASSES - 1)


def _loss_body(x_ref, t_ref, out_ref, *, rows, k):
    i = pl.program_id(0)
    x = x_ref[...]
    m = jnp.max(x, axis=1, keepdims=True)
    e = jnp.exp(x - m)
    lse = jnp.log(jnp.sum(e, axis=1, keepdims=True))
    sum_x = jnp.sum(x, axis=1, keepdims=True)
    sum_logp = sum_x - float(k) * (m + lse)

    t = t_ref[0, 0, :]
    col = jax.lax.broadcasted_iota(jnp.int32, (rows, k), 1)
    p_t = jnp.sum(jnp.where(col == t[:, None], x, 0.0), axis=1, keepdims=True)
    logp_t = p_t - m - lse

    row_loss = -(_SMOOTH_VAL * sum_logp + (_CONFIDENCE - _SMOOTH_VAL) * logp_t)
    partial = jnp.sum(row_loss).reshape(1, 1)

    @pl.when(i == 0)
    def _init():
        out_ref[...] = partial

    @pl.when(i != 0)
    def _acc():
        out_ref[...] += partial


def kernel(pred, target):
    n, k = pred.shape
    rows = 2048
    num_blocks = n // rows
    t3 = target.astype(jnp.int32).reshape(num_blocks, 1, rows)

    total = pl.pallas_call(
        functools.partial(_loss_body, rows=rows, k=k),
        grid=(num_blocks,),
        in_specs=[
            pl.BlockSpec((rows, k), lambda i: (i, 0)),
            pl.BlockSpec((1, 1, rows), lambda i: (i, 0, 0)),
        ],
        out_specs=pl.BlockSpec((1, 1), lambda i: (0, 0)),
        out_shape=jax.ShapeDtypeStruct((1, 1), jnp.float32),
    )(pred, t3)
    return (total[0, 0] / n).astype(jnp.float32)
